# baseline (speedup 1.0000x reference)
import functools

import jax
import jax.numpy as jnp
from jax import lax
from jax.experimental import pallas as pl
from jax.experimental.pallas import tpu as pltpu
from jax.experimental.pallas import tpu_sc as plsc

B = 1024
V = 100000
NS = 16
L = 16
PER = B // NS

_mesh = plsc.VectorSubcoreMesh(
    core_axis_name="c", subcore_axis_name="s", num_cores=1)


@functools.partial(
    pl.kernel,
    out_type=jax.ShapeDtypeStruct((NS * L,), jnp.float32),
    mesh=_mesh,
    compiler_params=pltpu.CompilerParams(needs_layout_passes=False),
    scratch_types=[
        pltpu.VMEM((PER,), jnp.int32),
        pltpu.VMEM((PER,), jnp.float32),
        pltpu.SemaphoreType.DMA,
    ],
)
def _nll_sc(flat_hbm, tgt_hbm, out_hbm, idx_v, vals_v, sem):
    sid = lax.axis_index("s")
    base = sid * PER

    pltpu.sync_copy(tgt_hbm.at[pl.ds(base, PER)], idx_v)
    for j in range(PER // L):
        t = idx_v[pl.ds(j * L, L)]
        rows = (base + j * L) + lax.iota(jnp.int32, L)
        idx_v[pl.ds(j * L, L)] = rows * V + t

    pltpu.async_copy(flat_hbm.at[idx_v], vals_v, sem).wait()

    part = vals_v[pl.ds(0, L)]
    for j in range(1, PER // L):
        part = part + vals_v[pl.ds(j * L, L)]
    vals_v[pl.ds(0, L)] = part
    pltpu.sync_copy(vals_v.at[pl.ds(0, L)], out_hbm.at[pl.ds(sid * L, L)])


def kernel(input_tensor, target_tensor):
    out = _nll_sc(input_tensor.reshape(-1), target_tensor.astype(jnp.int32))
    return -jnp.sum(out) / B
